# baseline (speedup 1.0000x reference)
import functools

import jax
import jax.numpy as jnp
from jax import lax
from jax.experimental import pallas as pl
from jax.experimental.pallas import tpu as pltpu
from jax.experimental.pallas import tpu_sc as plsc

B = 16384
C = 64
NC = 2
NS = 16
NW = NC * NS
BPW = B // NW
CHUNK = 128
NCH = BPW // CHUNK
L = 16
GPC = CHUNK // L


def _vsqrt(x):
    i = lax.bitcast_convert_type(x, jnp.int32)
    i = (i >> 1) + jnp.int32(0x1FBD1DF5)
    y = lax.bitcast_convert_type(i, jnp.float32)
    y = 0.5 * (y + x / y)
    y = 0.5 * (y + x / y)
    y = 0.5 * (y + x / y)
    return y


def _lanesum(v):
    acc = v[0]
    for i in range(1, L):
        acc = acc + v[i]
    return acc


def _body(user_hbm, item_hbm, ustruct_hbm, istruct_hbm, rh_hbm, rel_hbm,
          out_hbm, uidx_v, iidx_v, ug_v, ig_v, u128_v, i128_v, rh_v, rel_v,
          out_v, sem0, sem1):
    wid = lax.axis_index("s") * NC + lax.axis_index("c")
    base = wid * BPW

    for j in range(NCH):
        pltpu.sync_copy(user_hbm.at[pl.ds(base + j * CHUNK, CHUNK)],
                        uidx_v.at[j])
        pltpu.sync_copy(item_hbm.at[pl.ds(base + j * CHUNK, CHUNK)],
                        iidx_v.at[j])
    pltpu.sync_copy(rh_hbm, rh_v)
    pltpu.sync_copy(rel_hbm, rel_v)

    for j in range(NCH):
        for t in range(GPC):
            sl = pl.ds(t * L, L)
            ug_v[j, sl] = uidx_v[j, sl] >> 1
            ig_v[j, sl] = iidx_v[j, sl] >> 1

    rh = [rh_v[pl.ds(k * L, L)] for k in range(C // L)]
    rele = [rel_v[pl.ds(k * L, L)] + 1e-6 for k in range(C // L)]
    s = rh[0] * rh[0]
    p = rh[0] * rele[0]
    for k in range(1, C // L):
        s = s + rh[k] * rh[k]
        p = p + rh[k] * rele[k]
    n2 = _lanesum(s)
    n2v = jnp.full((L,), 0.0, jnp.float32) + n2
    invv = 1.0 / jnp.maximum(_vsqrt(n2v), 1e-12)
    inv = invv[0]
    g2 = n2 * inv * inv
    rho = _lanesum(p) * inv
    ca = 2.0 - g2
    cb = 2.0 * rho
    rhn = [r * inv for r in rh]

    sems = [sem0, sem1]

    def fire(j):
        slot = j % 2
        return [
            pltpu.async_copy(ustruct_hbm.at[ug_v.at[j]], u128_v.at[slot],
                             sems[slot]),
            pltpu.async_copy(istruct_hbm.at[ig_v.at[j]], i128_v.at[slot],
                             sems[slot]),
        ]

    inflight = {0: fire(0)}

    for j in range(NCH):
        if j + 1 < NCH:
            inflight[j + 1] = fire(j + 1)
        for cp in inflight.pop(j):
            cp.wait()
        slot = j % 2

        def group_body(t, _):
            lrvec = t * L + lax.iota(jnp.int32, L)
            sl = pl.ds(t * L, L)
            cu = (uidx_v[j, sl] & 1) * 64
            ci = (iidx_v[j, sl] & 1) * 64
            acc_a = jnp.zeros((L,), jnp.float32)
            acc_d = jnp.zeros((L,), jnp.float32)
            for c in range(C):
                u = plsc.load_gather(u128_v.at[slot], [lrvec, cu + c])
                it = plsc.load_gather(i128_v.at[slot], [lrvec, ci + c])
                d = u - it
                e = d + rele[c // L][c % L]
                acc_a = acc_a + e * e
                acc_d = acc_d + d * rhn[c // L][c % L]
            ssq = acc_a - ca * acc_d * acc_d - cb * acc_d
            out_v[pl.ds(j * CHUNK + t * L, L)] = _vsqrt(ssq)
            return 0

        lax.fori_loop(0, GPC, group_body, 0)

    pltpu.sync_copy(out_v, out_hbm.at[pl.ds(base, BPW)])


@jax.jit
def _transh(user, item, user_structure, item_structure, rh, rel):
    mesh = plsc.VectorSubcoreMesh(core_axis_name="c", subcore_axis_name="s")
    return pl.kernel(
        _body,
        out_type=jax.ShapeDtypeStruct((B,), jnp.float32),
        mesh=mesh,
        compiler_params=pltpu.CompilerParams(needs_layout_passes=False),
        scratch_types=[
            pltpu.VMEM((NCH, CHUNK), jnp.int32),
            pltpu.VMEM((NCH, CHUNK), jnp.int32),
            pltpu.VMEM((NCH, CHUNK), jnp.int32),
            pltpu.VMEM((NCH, CHUNK), jnp.int32),
            pltpu.VMEM((2, CHUNK, 2 * C), jnp.float32),
            pltpu.VMEM((2, CHUNK, 2 * C), jnp.float32),
            pltpu.VMEM((C,), jnp.float32),
            pltpu.VMEM((C,), jnp.float32),
            pltpu.VMEM((BPW,), jnp.float32),
            pltpu.SemaphoreType.DMA,
            pltpu.SemaphoreType.DMA,
        ],
    )(user, item, user_structure, item_structure, rh, rel)


def kernel(user, item, user_structure, item_structure, relation_embedding,
           relationHyper):
    rh = relationHyper.reshape(C)
    rel = relation_embedding.reshape(C)
    us2 = user_structure.reshape(user_structure.shape[0] // 2, 2 * C)
    is2 = item_structure.reshape(item_structure.shape[0] // 2, 2 * C)
    return _transh(user.astype(jnp.int32), item.astype(jnp.int32),
                   us2, is2, rh, rel)


# speedup vs baseline: 1.5564x; 1.5564x over previous
import functools

import jax
import jax.numpy as jnp
from jax import lax
from jax.experimental import pallas as pl
from jax.experimental.pallas import tpu as pltpu
from jax.experimental.pallas import tpu_sc as plsc

B = 16384
C = 64
NC = 2
NS = 16
NW = NC * NS
BPW = B // NW
L = 16
NG = BPW // L


def _vsqrt(x):
    i = lax.bitcast_convert_type(x, jnp.int32)
    i = (i >> 1) + jnp.int32(0x1FBD1DF5)
    y = lax.bitcast_convert_type(i, jnp.float32)
    y = 0.5 * (y + x / y)
    y = 0.5 * (y + x / y)
    y = 0.5 * (y + x / y)
    return y


def _lanesum(v):
    acc = v[0]
    for i in range(1, L):
        acc = acc + v[i]
    return acc


CHUNK = 128
NCH = BPW // CHUNK
GPC = CHUNK // L


def _body(user_hbm, item_hbm, ustruct_hbm, istruct_hbm, rh_hbm, rel_hbm,
          out_hbm, uidx_v, iidx_v, urows_v, irows_v, rh_v, rel_v, out_v,
          sem0, sem1):
    wid = lax.axis_index("s") * NC + lax.axis_index("c")
    base = wid * BPW

    pltpu.sync_copy(user_hbm.at[pl.ds(base, BPW)], uidx_v)
    pltpu.sync_copy(item_hbm.at[pl.ds(base, BPW)], iidx_v)
    pltpu.sync_copy(rh_hbm, rh_v)
    pltpu.sync_copy(rel_hbm, rel_v)

    sems = [sem0, sem1]

    def issue_chunk(j):
        slot = j % 2
        sem = sems[slot]

        def issue_body(t, _):
            uvec = uidx_v[pl.ds(j * CHUNK + t * L, L)]
            ivec = iidx_v[pl.ds(j * CHUNK + t * L, L)]
            for k in range(L):
                pltpu.async_copy(ustruct_hbm.at[uvec[k]],
                                 urows_v.at[slot].at[t * L + k], sem)
                pltpu.async_copy(istruct_hbm.at[ivec[k]],
                                 irows_v.at[slot].at[t * L + k], sem)
            return 0

        lax.fori_loop(0, GPC, issue_body, 0)

    def drain_chunk(j):
        slot = j % 2
        pltpu.make_async_copy(ustruct_hbm.at[pl.ds(0, CHUNK)],
                              urows_v.at[slot], sems[slot]).wait()
        pltpu.make_async_copy(istruct_hbm.at[pl.ds(0, CHUNK)],
                              irows_v.at[slot], sems[slot]).wait()

    issue_chunk(0)

    rh = [rh_v[pl.ds(k * L, L)] for k in range(C // L)]
    rele = [rel_v[pl.ds(k * L, L)] + 1e-6 for k in range(C // L)]
    s = rh[0] * rh[0]
    p = rh[0] * rele[0]
    for k in range(1, C // L):
        s = s + rh[k] * rh[k]
        p = p + rh[k] * rele[k]
    n2 = _lanesum(s)
    n2v = jnp.full((L,), 0.0, jnp.float32) + n2
    invv = 1.0 / jnp.maximum(_vsqrt(n2v), 1e-12)
    inv = invv[0]
    g2 = n2 * inv * inv
    rho = _lanesum(p) * inv
    ca = 2.0 - g2
    cb = 2.0 * rho
    rhn = [r * inv for r in rh]

    for j in range(NCH):
        if j + 1 < NCH:
            issue_chunk(j + 1)
        drain_chunk(j)
        slot = j % 2

        def group_body(t, _):
            rvec = t * L + lax.iota(jnp.int32, L)
            acc_a = jnp.zeros((L,), jnp.float32)
            acc_d = jnp.zeros((L,), jnp.float32)
            for c in range(C):
                cvec = jnp.full((L,), c, jnp.int32)
                u = plsc.load_gather(urows_v.at[slot], [rvec, cvec])
                it = plsc.load_gather(irows_v.at[slot], [rvec, cvec])
                d = u - it
                e = d + rele[c // L][c % L]
                acc_a = acc_a + e * e
                acc_d = acc_d + d * rhn[c // L][c % L]
            ssq = acc_a - ca * acc_d * acc_d - cb * acc_d
            out_v[pl.ds(j * CHUNK + t * L, L)] = _vsqrt(ssq)
            return 0

        lax.fori_loop(0, GPC, group_body, 0)

    pltpu.sync_copy(out_v, out_hbm.at[pl.ds(base, BPW)])


@jax.jit
def _transh(user, item, user_structure, item_structure, rh, rel):
    mesh = plsc.VectorSubcoreMesh(core_axis_name="c", subcore_axis_name="s")
    return pl.kernel(
        _body,
        out_type=jax.ShapeDtypeStruct((B,), jnp.float32),
        mesh=mesh,
        compiler_params=pltpu.CompilerParams(needs_layout_passes=False),
        scratch_types=[
            pltpu.VMEM((BPW,), jnp.int32),
            pltpu.VMEM((BPW,), jnp.int32),
            pltpu.VMEM((2, CHUNK, C), jnp.float32),
            pltpu.VMEM((2, CHUNK, C), jnp.float32),
            pltpu.VMEM((C,), jnp.float32),
            pltpu.VMEM((C,), jnp.float32),
            pltpu.VMEM((BPW,), jnp.float32),
            pltpu.SemaphoreType.DMA,
            pltpu.SemaphoreType.DMA,
        ],
    )(user, item, user_structure, item_structure, rh, rel)


def kernel(user, item, user_structure, item_structure, relation_embedding,
           relationHyper):
    rh = relationHyper.reshape(C)
    rel = relation_embedding.reshape(C)
    return _transh(user.astype(jnp.int32), item.astype(jnp.int32),
                   user_structure, item_structure, rh, rel)


# speedup vs baseline: 2.0504x; 1.3175x over previous
import functools

import jax
import jax.numpy as jnp
from jax import lax
from jax.experimental import pallas as pl
from jax.experimental.pallas import tpu as pltpu
from jax.experimental.pallas import tpu_sc as plsc

B = 16384
C = 64
NC = 2
NS = 16
NW = NC * NS
BPW = B // NW
L = 16
NG = BPW // L
NBUF = 3


def _vsqrt(x):
    i = lax.bitcast_convert_type(x, jnp.int32)
    i = (i >> 1) + jnp.int32(0x1FBD1DF5)
    y = lax.bitcast_convert_type(i, jnp.float32)
    y = 0.5 * (y + x / y)
    y = 0.5 * (y + x / y)
    y = 0.5 * (y + x / y)
    return y


def _lanesum(v):
    acc = v[0]
    for i in range(1, L):
        acc = acc + v[i]
    return acc


def _body(user_hbm, item_hbm, ustruct_hbm, istruct_hbm, rh_hbm, rel_hbm,
          out_hbm, uidx_v, iidx_v, ublk_v, iblk_v, urow_v, irow_v, rh_v,
          rel_v, out_v, sem0, sem1, sem2):
    wid = lax.axis_index("s") * NC + lax.axis_index("c")
    base = wid * BPW

    pltpu.sync_copy(user_hbm.at[pl.ds(base, BPW)], uidx_v)
    pltpu.sync_copy(item_hbm.at[pl.ds(base, BPW)], iidx_v)
    pltpu.sync_copy(rh_hbm, rh_v)
    pltpu.sync_copy(rel_hbm, rel_v)

    sems = [sem0, sem1, sem2]

    rh = [rh_v[pl.ds(k * L, L)] for k in range(C // L)]
    rele = [rel_v[pl.ds(k * L, L)] + 1e-6 for k in range(C // L)]
    s = rh[0] * rh[0]
    p = rh[0] * rele[0]
    for k in range(1, C // L):
        s = s + rh[k] * rh[k]
        p = p + rh[k] * rele[k]
    n2 = _lanesum(s)
    n2v = jnp.full((L,), 0.0, jnp.float32) + n2
    invv = 1.0 / jnp.maximum(_vsqrt(n2v), 1e-12)
    inv = invv[0]
    g2 = n2 * inv * inv
    rho = _lanesum(p) * inv
    ca = 2.0 - g2
    cb = 2.0 * rho
    rhn = [r * inv for r in rh]

    iota = lax.iota(jnp.int32, L)
    zero = jnp.zeros((L,), jnp.int32)

    def group_body(g, _):
        gsl = pl.ds(g * L, L)
        uvec = uidx_v[gsl]
        ivec = iidx_v[gsl]

        def issue(k, slot):
            offu = pl.multiple_of((uvec[k] >> 7) * 128, 128)
            offi = pl.multiple_of((ivec[k] >> 7) * 128, 128)
            pltpu.async_copy(ustruct_hbm.at[:, pl.ds(offu, 128)],
                             ublk_v.at[slot], sems[slot])
            pltpu.async_copy(istruct_hbm.at[:, pl.ds(offi, 128)],
                             iblk_v.at[slot], sems[slot])

        def drain(slot):
            pltpu.make_async_copy(ustruct_hbm.at[:, pl.ds(0, 128)],
                                  ublk_v.at[slot], sems[slot]).wait()
            pltpu.make_async_copy(istruct_hbm.at[:, pl.ds(0, 128)],
                                  iblk_v.at[slot], sems[slot]).wait()

        def extract(k, slot):
            lu = zero + (uvec[k] & 127)
            li = zero + (ivec[k] & 127)
            for k4 in range(C // L):
                fv = k4 * L + iota
                urow_v[k, pl.ds(k4 * L, L)] = plsc.load_gather(
                    ublk_v.at[slot], [fv, lu])
                irow_v[k, pl.ds(k4 * L, L)] = plsc.load_gather(
                    iblk_v.at[slot], [fv, li])

        issue(0, 0)
        issue(1, 1)
        for k in range(L):
            drain(k % 3)
            extract(k, k % 3)
            if k + 2 < L:
                issue(k + 2, (k + 2) % 3)

        acc_a = jnp.zeros((L,), jnp.float32)
        acc_d = jnp.zeros((L,), jnp.float32)
        for c in range(C):
            cvec = zero + c
            u = plsc.load_gather(urow_v, [iota, cvec])
            it = plsc.load_gather(irow_v, [iota, cvec])
            d = u - it
            e = d + rele[c // L][c % L]
            acc_a = acc_a + e * e
            acc_d = acc_d + d * rhn[c // L][c % L]
        ssq = acc_a - ca * acc_d * acc_d - cb * acc_d
        out_v[gsl] = _vsqrt(ssq)
        return 0

    lax.fori_loop(0, NG, group_body, 0)

    pltpu.sync_copy(out_v, out_hbm.at[pl.ds(base, BPW)])


@jax.jit
def _transh(user, item, user_structure_t, item_structure_t, rh, rel):
    mesh = plsc.VectorSubcoreMesh(core_axis_name="c", subcore_axis_name="s")
    return pl.kernel(
        _body,
        out_type=jax.ShapeDtypeStruct((B,), jnp.float32),
        mesh=mesh,
        compiler_params=pltpu.CompilerParams(needs_layout_passes=False),
        scratch_types=[
            pltpu.VMEM((BPW,), jnp.int32),
            pltpu.VMEM((BPW,), jnp.int32),
            pltpu.VMEM((NBUF, C, 128), jnp.float32),
            pltpu.VMEM((NBUF, C, 128), jnp.float32),
            pltpu.VMEM((L, C), jnp.float32),
            pltpu.VMEM((L, C), jnp.float32),
            pltpu.VMEM((C,), jnp.float32),
            pltpu.VMEM((C,), jnp.float32),
            pltpu.VMEM((BPW,), jnp.float32),
            pltpu.SemaphoreType.DMA,
            pltpu.SemaphoreType.DMA,
            pltpu.SemaphoreType.DMA,
        ],
    )(user, item, user_structure_t, item_structure_t, rh, rel)


def kernel(user, item, user_structure, item_structure, relation_embedding,
           relationHyper):
    rh = relationHyper.reshape(C)
    rel = relation_embedding.reshape(C)
    return _transh(user.astype(jnp.int32), item.astype(jnp.int32),
                   user_structure.T, item_structure.T, rh, rel)


# speedup vs baseline: 2.1959x; 1.0709x over previous
import functools

import jax
import jax.numpy as jnp
from jax import lax
from jax.experimental import pallas as pl
from jax.experimental.pallas import tpu as pltpu
from jax.experimental.pallas import tpu_sc as plsc

B = 16384
C = 64
NC = 2
NS = 16
NW = NC * NS
BPW = B // NW
L = 16
NG = BPW // L
NBUF = 2


def _vsqrt(x):
    i = lax.bitcast_convert_type(x, jnp.int32)
    i = (i >> 1) + jnp.int32(0x1FBD1DF5)
    y = lax.bitcast_convert_type(i, jnp.float32)
    y = 0.5 * (y + x / y)
    y = 0.5 * (y + x / y)
    y = 0.5 * (y + x / y)
    return y


def _lanesum(v):
    acc = v[0]
    for i in range(1, L):
        acc = acc + v[i]
    return acc


def _body(user_hbm, item_hbm, ustruct_hbm, istruct_hbm, rh_hbm, rel_hbm,
          out_hbm, uidx_v, iidx_v, ublk_v, iblk_v, urow_v, irow_v, rh_v,
          rel_v, out_v, sem0, sem1):
    wid = lax.axis_index("s") * NC + lax.axis_index("c")
    base = wid * BPW

    pltpu.sync_copy(user_hbm.at[pl.ds(base, BPW)], uidx_v)
    pltpu.sync_copy(item_hbm.at[pl.ds(base, BPW)], iidx_v)
    pltpu.sync_copy(rh_hbm, rh_v)
    pltpu.sync_copy(rel_hbm, rel_v)

    sems = [sem0, sem1]

    rh = [rh_v[pl.ds(k * L, L)] for k in range(C // L)]
    rele = [rel_v[pl.ds(k * L, L)] + 1e-6 for k in range(C // L)]
    s = rh[0] * rh[0]
    p = rh[0] * rele[0]
    for k in range(1, C // L):
        s = s + rh[k] * rh[k]
        p = p + rh[k] * rele[k]
    n2 = _lanesum(s)
    n2v = jnp.full((L,), 0.0, jnp.float32) + n2
    invv = 1.0 / jnp.maximum(_vsqrt(n2v), 1e-12)
    inv = invv[0]
    g2 = n2 * inv * inv
    rho = _lanesum(p) * inv
    ca = 2.0 - g2
    cb = 2.0 * rho
    rhn = [r * inv for r in rh]

    iota = lax.iota(jnp.int32, L)
    zero = jnp.zeros((L,), jnp.int32)

    def issue(uscalar, iscalar, slot):
        offu = pl.multiple_of((uscalar >> 7) * 128, 128)
        offi = pl.multiple_of((iscalar >> 7) * 128, 128)
        pltpu.async_copy(ustruct_hbm.at[:, pl.ds(offu, 128)],
                         ublk_v.at[slot], sems[slot])
        pltpu.async_copy(istruct_hbm.at[:, pl.ds(offi, 128)],
                         iblk_v.at[slot], sems[slot])

    def drain(slot):
        pltpu.make_async_copy(ustruct_hbm.at[:, pl.ds(0, 128)],
                              ublk_v.at[slot], sems[slot]).wait()
        pltpu.make_async_copy(istruct_hbm.at[:, pl.ds(0, 128)],
                              iblk_v.at[slot], sems[slot]).wait()

    uvec0 = uidx_v[pl.ds(0, L)]
    ivec0 = iidx_v[pl.ds(0, L)]
    issue(uvec0[0], ivec0[0], 0)
    issue(uvec0[1], ivec0[1], 1)

    def group_body(g, _):
        gsl = pl.ds(g * L, L)
        uvec = uidx_v[gsl]
        ivec = iidx_v[gsl]
        nsl = pl.ds(((g + 1) % NG) * L, L)
        uvn = uidx_v[nsl]
        ivn = iidx_v[nsl]

        def extract(k, slot):
            lu = zero + (uvec[k] & 127)
            li = zero + (ivec[k] & 127)
            for k4 in range(C // L):
                fv = k4 * L + iota
                urow_v[k, pl.ds(k4 * L, L)] = plsc.load_gather(
                    ublk_v.at[slot], [fv, lu])
                irow_v[k, pl.ds(k4 * L, L)] = plsc.load_gather(
                    iblk_v.at[slot], [fv, li])

        for k in range(L):
            drain(k % 2)
            extract(k, k % 2)
            if k + 2 < L:
                issue(uvec[k + 2], ivec[k + 2], k % 2)
            else:
                issue(uvn[k + 2 - L], ivn[k + 2 - L], k % 2)

        acc_a = jnp.zeros((L,), jnp.float32)
        acc_d = jnp.zeros((L,), jnp.float32)
        for c in range(C):
            cvec = zero + c
            u = plsc.load_gather(urow_v, [iota, cvec])
            it = plsc.load_gather(irow_v, [iota, cvec])
            d = u - it
            e = d + rele[c // L][c % L]
            acc_a = acc_a + e * e
            acc_d = acc_d + d * rhn[c // L][c % L]
        ssq = acc_a - ca * acc_d * acc_d - cb * acc_d
        out_v[gsl] = _vsqrt(ssq)
        return 0

    lax.fori_loop(0, NG, group_body, 0)

    drain(0)
    drain(1)

    pltpu.sync_copy(out_v, out_hbm.at[pl.ds(base, BPW)])


@jax.jit
def _transh(user, item, user_structure_t, item_structure_t, rh, rel):
    mesh = plsc.VectorSubcoreMesh(core_axis_name="c", subcore_axis_name="s")
    return pl.kernel(
        _body,
        out_type=jax.ShapeDtypeStruct((B,), jnp.float32),
        mesh=mesh,
        compiler_params=pltpu.CompilerParams(needs_layout_passes=False),
        scratch_types=[
            pltpu.VMEM((BPW,), jnp.int32),
            pltpu.VMEM((BPW,), jnp.int32),
            pltpu.VMEM((NBUF, C, 128), jnp.float32),
            pltpu.VMEM((NBUF, C, 128), jnp.float32),
            pltpu.VMEM((L, C), jnp.float32),
            pltpu.VMEM((L, C), jnp.float32),
            pltpu.VMEM((C,), jnp.float32),
            pltpu.VMEM((C,), jnp.float32),
            pltpu.VMEM((BPW,), jnp.float32),
            pltpu.SemaphoreType.DMA,
            pltpu.SemaphoreType.DMA,
        ],
    )(user, item, user_structure_t, item_structure_t, rh, rel)


def kernel(user, item, user_structure, item_structure, relation_embedding,
           relationHyper):
    rh = relationHyper.reshape(C)
    rel = relation_embedding.reshape(C)
    return _transh(user.astype(jnp.int32), item.astype(jnp.int32),
                   user_structure.T, item_structure.T, rh, rel)


# speedup vs baseline: 2.3736x; 1.0809x over previous
import functools

import jax
import jax.numpy as jnp
from jax import lax
from jax.experimental import pallas as pl
from jax.experimental.pallas import tpu as pltpu
from jax.experimental.pallas import tpu_sc as plsc

B = 16384
C = 64
NC = 2
NS = 16
NW = NC * NS
BPW = B // NW
L = 16
NG = BPW // L
NBUF = 2


def _vsqrt(x):
    i = lax.bitcast_convert_type(x, jnp.int32)
    i = (i >> 1) + jnp.int32(0x1FBD1DF5)
    y = lax.bitcast_convert_type(i, jnp.float32)
    y = 0.5 * (y + x / y)
    y = 0.5 * (y + x / y)
    y = 0.5 * (y + x / y)
    return y


def _lanesum(v):
    acc = v[0]
    for i in range(1, L):
        acc = acc + v[i]
    return acc


def _body(user_hbm, item_hbm, ustruct_hbm, istruct_hbm, rh_hbm, rel_hbm,
          out_hbm, uidx_v, iidx_v, ublk_v, iblk_v, urow_v, irow_v, rh_v,
          rel_v, out_v, sem0, sem1):
    wid = lax.axis_index("s") * NC + lax.axis_index("c")
    base = wid * BPW

    pltpu.sync_copy(user_hbm.at[pl.ds(base, BPW)], uidx_v)
    pltpu.sync_copy(item_hbm.at[pl.ds(base, BPW)], iidx_v)
    pltpu.sync_copy(rh_hbm, rh_v)
    pltpu.sync_copy(rel_hbm, rel_v)

    sems = [sem0, sem1]

    rh = [rh_v[pl.ds(k * L, L)] for k in range(C // L)]
    rele = [rel_v[pl.ds(k * L, L)] + 1e-6 for k in range(C // L)]
    s = rh[0] * rh[0]
    p = rh[0] * rele[0]
    for k in range(1, C // L):
        s = s + rh[k] * rh[k]
        p = p + rh[k] * rele[k]
    n2 = _lanesum(s)
    n2v = jnp.full((L,), 0.0, jnp.float32) + n2
    invv = 1.0 / jnp.maximum(_vsqrt(n2v), 1e-12)
    inv = invv[0]
    g2 = n2 * inv * inv
    rho = _lanesum(p) * inv
    ca = 2.0 - g2
    cb = 2.0 * rho
    rhn = [r * inv for r in rh]

    iota = lax.iota(jnp.int32, L)
    zero = jnp.zeros((L,), jnp.int32)

    def issue_pair(uvecs, ivecs, k0, slot):
        for h in range(2):
            offu = pl.multiple_of((uvecs[k0 + h] >> 7) * 128, 128)
            offi = pl.multiple_of((ivecs[k0 + h] >> 7) * 128, 128)
            hs = pl.ds(h * 128, 128)
            pltpu.async_copy(ustruct_hbm.at[:, pl.ds(offu, 128)],
                             ublk_v.at[slot].at[:, hs], sems[slot])
            pltpu.async_copy(istruct_hbm.at[:, pl.ds(offi, 128)],
                             iblk_v.at[slot].at[:, hs], sems[slot])

    def drain(slot):
        pltpu.make_async_copy(ustruct_hbm.at[:, pl.ds(0, 256)],
                              ublk_v.at[slot], sems[slot]).wait()
        pltpu.make_async_copy(istruct_hbm.at[:, pl.ds(0, 256)],
                              iblk_v.at[slot], sems[slot]).wait()

    uvec0 = uidx_v[pl.ds(0, L)]
    ivec0 = iidx_v[pl.ds(0, L)]
    issue_pair(uvec0, ivec0, 0, 0)
    issue_pair(uvec0, ivec0, 2, 1)

    def group_body(g, _):
        gsl = pl.ds(g * L, L)
        uvec = uidx_v[gsl]
        ivec = iidx_v[gsl]
        nsl = pl.ds(((g + 1) % NG) * L, L)
        uvn = uidx_v[nsl]
        ivn = iidx_v[nsl]

        def extract(k, slot, half):
            lu = (zero + (uvec[k] & 127)) + half * 128
            li = (zero + (ivec[k] & 127)) + half * 128
            for k4 in range(C // L):
                fv = k4 * L + iota
                urow_v[k, pl.ds(k4 * L, L)] = plsc.load_gather(
                    ublk_v.at[slot], [fv, lu])
                irow_v[k, pl.ds(k4 * L, L)] = plsc.load_gather(
                    iblk_v.at[slot], [fv, li])

        for m in range(L // 2):
            ss = m % 2
            drain(ss)
            extract(2 * m, ss, 0)
            extract(2 * m + 1, ss, 1)
            if 2 * m + 4 < L:
                issue_pair(uvec, ivec, 2 * m + 4, ss)
            else:
                issue_pair(uvn, ivn, 2 * m + 4 - L, ss)

        acc_a = jnp.zeros((L,), jnp.float32)
        acc_d = jnp.zeros((L,), jnp.float32)
        for c in range(C):
            cvec = zero + c
            u = plsc.load_gather(urow_v, [iota, cvec])
            it = plsc.load_gather(irow_v, [iota, cvec])
            d = u - it
            e = d + rele[c // L][c % L]
            acc_a = acc_a + e * e
            acc_d = acc_d + d * rhn[c // L][c % L]
        ssq = acc_a - ca * acc_d * acc_d - cb * acc_d
        out_v[gsl] = _vsqrt(ssq)
        return 0

    lax.fori_loop(0, NG, group_body, 0)

    drain(0)
    drain(1)

    pltpu.sync_copy(out_v, out_hbm.at[pl.ds(base, BPW)])


@jax.jit
def _transh(user, item, user_structure_t, item_structure_t, rh, rel):
    mesh = plsc.VectorSubcoreMesh(core_axis_name="c", subcore_axis_name="s")
    return pl.kernel(
        _body,
        out_type=jax.ShapeDtypeStruct((B,), jnp.float32),
        mesh=mesh,
        compiler_params=pltpu.CompilerParams(needs_layout_passes=False),
        scratch_types=[
            pltpu.VMEM((BPW,), jnp.int32),
            pltpu.VMEM((BPW,), jnp.int32),
            pltpu.VMEM((NBUF, C, 256), jnp.float32),
            pltpu.VMEM((NBUF, C, 256), jnp.float32),
            pltpu.VMEM((L, C), jnp.float32),
            pltpu.VMEM((L, C), jnp.float32),
            pltpu.VMEM((C,), jnp.float32),
            pltpu.VMEM((C,), jnp.float32),
            pltpu.VMEM((BPW,), jnp.float32),
            pltpu.SemaphoreType.DMA,
            pltpu.SemaphoreType.DMA,
        ],
    )(user, item, user_structure_t, item_structure_t, rh, rel)


def kernel(user, item, user_structure, item_structure, relation_embedding,
           relationHyper):
    rh = relationHyper.reshape(C)
    rel = relation_embedding.reshape(C)
    return _transh(user.astype(jnp.int32), item.astype(jnp.int32),
                   user_structure.T, item_structure.T, rh, rel)
